# baseline (speedup 1.0000x reference)
import functools

import jax
import jax.numpy as jnp
from jax import lax
from jax.experimental import pallas as pl
from jax.experimental.pallas import tpu as pltpu
from jax.experimental.pallas import tpu_sc as plsc

_NC = 2
_NS = 16
_NW = _NC * _NS
_T = 100
_TP = 104
_STYLE_DIM = 128


def _sc_body(idx_hbm, tab_hbm, out_hbm, idx_v, rows_v, sem):
    n_slabs = out_hbm.shape[0]
    per_w = n_slabs // _NW
    w = lax.axis_index("s") * _NC + lax.axis_index("c")
    base = w * per_w

    def step(j, carry):
        i = base + j
        pltpu.sync_copy(idx_hbm.at[pl.ds(i * _TP, _TP)], idx_v)
        pltpu.async_copy(tab_hbm.at[idx_v], rows_v, sem).wait()
        pltpu.sync_copy(rows_v.at[pl.ds(0, _T)], out_hbm.at[i])
        return carry

    lax.fori_loop(0, per_w, step, 0)


def kernel(indices, table):
    n = indices.shape[0]
    idx_pad = jnp.pad(indices.astype(jnp.int32), ((0, 0), (0, _TP - _T)))
    idx_flat = idx_pad.reshape(-1)
    mesh = plsc.VectorSubcoreMesh(core_axis_name="c", subcore_axis_name="s")
    f = functools.partial(
        pl.kernel,
        out_type=jax.ShapeDtypeStruct((n, _T, _STYLE_DIM), jnp.float32),
        mesh=mesh,
        scratch_types=[
            pltpu.VMEM((_TP,), jnp.int32),
            pltpu.VMEM((_TP, _STYLE_DIM), jnp.float32),
            pltpu.SemaphoreType.DMA,
        ],
    )(_sc_body)
    return f(idx_flat, table)
